# speedup vs baseline: 1.7637x; 1.7637x over previous
import functools

import jax
import jax.numpy as jnp
from jax import lax
from jax.experimental import pallas as pl
from jax.experimental.pallas import tpu as pltpu
from jax.experimental.pallas import tpu_sc as plsc


def _sc_gather(table, flat_idx, num_indices, feat):
    NC, NS = 2, 16
    NW = NC * NS
    assert num_indices % (8 * NW) == 0
    b_per_w = num_indices // NW
    chunk = 400
    assert b_per_w % chunk == 0 and chunk % 8 == 0
    n_chunks = b_per_w // chunk

    mesh = plsc.VectorSubcoreMesh(core_axis_name="c", subcore_axis_name="s")

    @functools.partial(
        pl.kernel,
        mesh=mesh,
        out_type=jax.ShapeDtypeStruct((num_indices, feat), jnp.float32),
        scratch_types=[
            pltpu.VMEM((chunk,), jnp.int32),
            pltpu.VMEM((chunk, feat), jnp.float32),
            pltpu.SemaphoreType.DMA,
        ],
    )
    def gather_kernel(table_hbm, idx_hbm, out_hbm, idx_v, rows_v, sem):
        wid = lax.axis_index("s") * NC + lax.axis_index("c")
        base = wid * b_per_w
        for k in range(n_chunks):
            off = base + k * chunk
            pltpu.sync_copy(idx_hbm.at[pl.ds(off, chunk)], idx_v)
            pltpu.async_copy(table_hbm.at[idx_v], rows_v, sem).wait()
            pltpu.sync_copy(rows_v, out_hbm.at[pl.ds(off, chunk)])

    return gather_kernel(table, flat_idx)


def _tc_softmax(x, num_rows, feat):
    block_rows = 1024
    assert num_rows % block_rows == 0

    def body(x_ref, o_ref):
        v = x_ref[...]
        m = jnp.max(v, axis=-1, keepdims=True)
        e = jnp.exp(v - m)
        o_ref[...] = e / jnp.sum(e, axis=-1, keepdims=True)

    return pl.pallas_call(
        body,
        out_shape=jax.ShapeDtypeStruct((num_rows, feat), jnp.float32),
        grid=(num_rows // block_rows,),
        in_specs=[pl.BlockSpec((block_rows, feat), lambda i: (i, 0))],
        out_specs=pl.BlockSpec((block_rows, feat), lambda i: (i, 0)),
    )(x)


def kernel(venueid2coor, inputs_wekn, poi_freq_matrix):
    del venueid2coor
    B, L = inputs_wekn.shape
    N, F = poi_freq_matrix.shape
    num_indices = B * L
    flat_idx = inputs_wekn.reshape(num_indices)

    gathered = _sc_gather(poi_freq_matrix, flat_idx, num_indices, F)
    out = _tc_softmax(gathered, num_indices, F)
    return out.reshape(B, L, F)
